# baseline (speedup 1.0000x reference)
import functools

import jax
import jax.numpy as jnp
from jax.experimental import pallas as pl
from jax.experimental.pallas import tpu as pltpu

_TB = 2048
_RC = 32
_ROTS = ((13, 15, 26, 6), (17, 29, 16, 24))


def _head_kernel(key_ref, x_ref, w_ref, b_ref, o_ref, xd_ref, *,
                 tb, h, thr, inv_keep):
    k1 = key_ref[0]
    k2 = key_ref[1]
    ks = (k1, k2, k1 ^ k2 ^ jnp.uint32(0x1BD11BDA))

    base = (pl.program_id(0) * (tb * h)).astype(jnp.uint32)
    local_k2 = (jax.lax.broadcasted_iota(jnp.uint32, (_RC, h), 0)
                * jnp.uint32(h)
                + jax.lax.broadcasted_iota(jnp.uint32, (_RC, h), 1)
                + k2)

    def threefry_store(j):
        r0 = j * _RC
        x1 = local_k2 + (base + jnp.uint32(j * (_RC * h)))
        x0 = x1 + k1
        x1 = (jax.lax.shift_left(x1, jnp.uint32(13))
              | jax.lax.shift_right_logical(x1, jnp.uint32(19))) ^ x0
        for i in range(5):
            for n, r in enumerate(_ROTS[i % 2]):
                if i == 0 and n == 0:
                    continue
                x0 = x0 + x1
                x1 = (jax.lax.shift_left(x1, jnp.uint32(r))
                      | jax.lax.shift_right_logical(x1, jnp.uint32(32 - r))
                      ) ^ x0
            x0 = x0 + ks[(i + 1) % 3]
            x1 = x1 + (ks[(i + 2) % 3] + jnp.uint32(i + 1))
        bits = x0 ^ x1
        scale = jnp.where(bits < jnp.uint32(thr),
                          jnp.float32(0.0), jnp.float32(inv_keep))
        xd_ref[pl.ds(r0, _RC), :] = x_ref[pl.ds(r0, _RC), :] * scale

    def chunk(j, carry):
        threefry_store(j)
        return carry

    nch = tb // _RC
    head = (nch - 2) * _RC
    jax.lax.fori_loop(0, nch - 2, chunk, 0, unroll=4)
    threefry_store(nch - 2)
    threefry_store(nch - 1)

    acc = jax.lax.dot_general(
        xd_ref[pl.ds(0, head), :], w_ref[...],
        dimension_numbers=(((1,), (1,)), ((), ())),
        preferred_element_type=jnp.float32)
    o_ref[pl.ds(0, head), :] = acc + b_ref[...][None, :]
    acc_t = jax.lax.dot_general(
        xd_ref[pl.ds(head, 2 * _RC), :], w_ref[...],
        dimension_numbers=(((1,), (1,)), ((), ())),
        preferred_element_type=jnp.float32)
    o_ref[pl.ds(head, 2 * _RC), :] = acc_t + b_ref[...][None, :]


def kernel(pooled_cls, weight, bias, rng_key):
    B, H = pooled_cls.shape
    num_labels = weight.shape[0]
    p = 0.3
    thr = min(int(p * 4294967296.0), 4294967295)
    inv_keep = 1.0 / (1.0 - p)

    tb = _TB if B % _TB == 0 else B
    x = pooled_cls.astype(jnp.float32)
    w = weight.astype(jnp.float32)
    b = bias.astype(jnp.float32)
    key = rng_key.astype(jnp.uint32)

    grid = (B // tb,)
    kernel_fn = functools.partial(
        _head_kernel, tb=tb, h=H, thr=thr, inv_keep=inv_keep)
    out = pl.pallas_call(
        kernel_fn,
        out_shape=jax.ShapeDtypeStruct((B, num_labels), jnp.float32),
        grid=grid,
        in_specs=[
            pl.BlockSpec(memory_space=pltpu.SMEM),
            pl.BlockSpec((tb, H), lambda i: (i, 0)),
            pl.BlockSpec((num_labels, H), lambda i: (0, 0)),
            pl.BlockSpec((num_labels,), lambda i: (0,)),
        ],
        out_specs=pl.BlockSpec((tb, num_labels), lambda i: (i, 0)),
        scratch_shapes=[pltpu.VMEM((tb, H), jnp.float32)],
        compiler_params=pltpu.CompilerParams(
            dimension_semantics=("parallel",)),
    )(key, x, w, b)
    return out


# speedup vs baseline: 1.0177x; 1.0177x over previous
import functools

import jax
import jax.numpy as jnp
from jax.experimental import pallas as pl
from jax.experimental.pallas import tpu as pltpu

_TB = 1024
_RC = 32
_ROTS = ((13, 15, 26, 6), (17, 29, 16, 24))


def _head_kernel(key_ref, x_ref, w_ref, b_ref, o_ref, xd_ref, *,
                 tb, h, thr, inv_keep):
    k1 = key_ref[0]
    k2 = key_ref[1]
    ks = (k1, k2, k1 ^ k2 ^ jnp.uint32(0x1BD11BDA))

    base = (pl.program_id(0) * (tb * h)).astype(jnp.uint32)
    local_k2 = (jax.lax.broadcasted_iota(jnp.uint32, (_RC, h), 0)
                * jnp.uint32(h)
                + jax.lax.broadcasted_iota(jnp.uint32, (_RC, h), 1)
                + k2)

    def threefry_store(j):
        r0 = j * _RC
        x1 = local_k2 + (base + jnp.uint32(j * (_RC * h)))
        x0 = x1 + k1
        x1 = (jax.lax.shift_left(x1, jnp.uint32(13))
              | jax.lax.shift_right_logical(x1, jnp.uint32(19))) ^ x0
        for i in range(5):
            for n, r in enumerate(_ROTS[i % 2]):
                if i == 0 and n == 0:
                    continue
                x0 = x0 + x1
                x1 = (jax.lax.shift_left(x1, jnp.uint32(r))
                      | jax.lax.shift_right_logical(x1, jnp.uint32(32 - r))
                      ) ^ x0
            x0 = x0 + ks[(i + 1) % 3]
            x1 = x1 + (ks[(i + 2) % 3] + jnp.uint32(i + 1))
        bits = x0 ^ x1
        scale = jnp.where(bits < jnp.uint32(thr),
                          jnp.float32(0.0), jnp.float32(inv_keep))
        xd_ref[pl.ds(r0, _RC), :] = x_ref[pl.ds(r0, _RC), :] * scale

    def chunk(j, carry):
        threefry_store(j)
        return carry

    nch = tb // _RC
    head = (nch - 2) * _RC
    jax.lax.fori_loop(0, nch - 2, chunk, 0, unroll=8)
    threefry_store(nch - 2)
    threefry_store(nch - 1)

    acc = jax.lax.dot_general(
        xd_ref[pl.ds(0, head), :], w_ref[...],
        dimension_numbers=(((1,), (1,)), ((), ())),
        preferred_element_type=jnp.float32)
    o_ref[pl.ds(0, head), :] = acc + b_ref[...][None, :]
    acc_t = jax.lax.dot_general(
        xd_ref[pl.ds(head, 2 * _RC), :], w_ref[...],
        dimension_numbers=(((1,), (1,)), ((), ())),
        preferred_element_type=jnp.float32)
    o_ref[pl.ds(head, 2 * _RC), :] = acc_t + b_ref[...][None, :]


def kernel(pooled_cls, weight, bias, rng_key):
    B, H = pooled_cls.shape
    num_labels = weight.shape[0]
    p = 0.3
    thr = min(int(p * 4294967296.0), 4294967295)
    inv_keep = 1.0 / (1.0 - p)

    tb = _TB if B % _TB == 0 else B
    x = pooled_cls.astype(jnp.float32)
    w = weight.astype(jnp.float32)
    b = bias.astype(jnp.float32)
    key = rng_key.astype(jnp.uint32)

    grid = (B // tb,)
    kernel_fn = functools.partial(
        _head_kernel, tb=tb, h=H, thr=thr, inv_keep=inv_keep)
    out = pl.pallas_call(
        kernel_fn,
        out_shape=jax.ShapeDtypeStruct((B, num_labels), jnp.float32),
        grid=grid,
        in_specs=[
            pl.BlockSpec(memory_space=pltpu.SMEM),
            pl.BlockSpec((tb, H), lambda i: (i, 0)),
            pl.BlockSpec((num_labels, H), lambda i: (0, 0)),
            pl.BlockSpec((num_labels,), lambda i: (0,)),
        ],
        out_specs=pl.BlockSpec((tb, num_labels), lambda i: (i, 0)),
        scratch_shapes=[pltpu.VMEM((tb, H), jnp.float32)],
        compiler_params=pltpu.CompilerParams(
            dimension_semantics=("parallel",)),
    )(key, x, w, b)
    return out
